# baseline (speedup 1.0000x reference)
import functools

import jax
import jax.numpy as jnp
from jax import lax
from jax.experimental import pallas as pl
from jax.experimental.pallas import tpu as pltpu
from jax.experimental.pallas import tpu_sc as plsc

B, L = 1024, 200
TOK = B * L
D = 64
NC, NS = 2, 16
NW = NC * NS
TPW = TOK // NW
ROWS_PW = TPW // L
SPLITS = ((0, 128), (128, 72))
NREP = NW


def _project_table(we_table, we_proj_w, we_proj_b):
    V, K = we_table.shape
    Dp = we_proj_w.shape[1]
    BM = 2048

    def mm(x_ref, w_ref, b_ref, o_ref):
        o_ref[...] = (
            jnp.dot(x_ref[...], w_ref[...], preferred_element_type=jnp.float32)
            + b_ref[...]
        )

    return pl.pallas_call(
        mm,
        grid=(pl.cdiv(V, BM),),
        in_specs=[
            pl.BlockSpec((BM, K), lambda i: (i, 0)),
            pl.BlockSpec((K, Dp), lambda i: (0, 0)),
            pl.BlockSpec((1, Dp), lambda i: (0, 0)),
        ],
        out_specs=pl.BlockSpec((BM, Dp), lambda i: (i, 0)),
        out_shape=jax.ShapeDtypeStruct((V, Dp), jnp.float32),
    )(we_table, we_proj_w, we_proj_b.reshape(1, Dp))


def _replicate(flat, n):
    m = flat.shape[0]

    def cp(x_ref, o_ref):
        o_ref[...] = x_ref[...]

    return pl.pallas_call(
        cp,
        grid=(n,),
        in_specs=[pl.BlockSpec((m,), lambda i: (0,))],
        out_specs=pl.BlockSpec((m,), lambda i: (i,)),
        out_shape=jax.ShapeDtypeStruct((n * m,), jnp.float32),
    )(flat)


def _transpose_out(flat2d):
    M, N = flat2d.shape
    BB, BK = 256, 2560

    def tr(x_ref, o_ref):
        o_ref[...] = x_ref[...].T

    return pl.pallas_call(
        tr,
        grid=(M // BB, N // BK),
        in_specs=[pl.BlockSpec((BB, BK), lambda i, j: (i, j))],
        out_specs=pl.BlockSpec((BK, BB), lambda i, j: (j, i)),
        out_shape=jax.ShapeDtypeStruct((N, M), jnp.float32),
    )(flat2d)


def _gather_sum(idx_list, proj, rep):
    mesh = plsc.VectorSubcoreMesh(core_axis_name="c", subcore_axis_name="s")

    @functools.partial(
        pl.kernel,
        out_type=jax.ShapeDtypeStruct((B, L, D), jnp.float32),
        mesh=mesh,
        scratch_types=[
            pltpu.VMEM((2, 10, L), jnp.int32),
            pltpu.VMEM((2, L, D), jnp.float32),
            pltpu.SemaphoreType.DMA((2,)),
            pltpu.SemaphoreType.DMA((2,)),
            pltpu.SemaphoreType.DMA((2,)),
        ],
        compiler_params=pltpu.CompilerParams(use_tc_tiling_on_sc=False),
    )
    def sc(i0, i1, i2, i3, i4, i5, i6, i7, i8, i9, proj_hbm, rep_hbm, out_hbm,
           idx_v, acc_v, sem_idx, sem_g, sem_out):
        wid = lax.axis_index("c") * NS + lax.axis_index("s")
        base = wid * TPW
        idx_hbms = (i0, i1, i2, i3, i4, i5, i6, i7, i8, i9)
        my_rep = rep_hbm.at[wid]
        tables = (proj_hbm,) + (my_rep,) * 9

        def fire_idx(r, p):
            off = base + r * L
            for t, ih in enumerate(idx_hbms):
                pltpu.async_copy(ih.at[pl.ds(off, L)],
                                 idx_v.at[p, t], sem_idx.at[p])

        def wait_idx(p):
            for t, ih in enumerate(idx_hbms):
                pltpu.make_async_copy(ih.at[pl.ds(base, L)],
                                      idx_v.at[p, t], sem_idx.at[p]).wait()

        def zero_acc(p):
            z = jnp.zeros((16,), jnp.float32)

            @pl.loop(0, L, unroll=8)
            def _row(i):
                for j in range(D // 16):
                    acc_v[p, i, pl.ds(j * 16, 16)] = z

        def fire_gathers(p):
            for lo, n in SPLITS:
                for t, tab in enumerate(tables):
                    pltpu.async_copy(
                        tab.at[idx_v.at[p, t, pl.ds(lo, n)]],
                        acc_v.at[p, pl.ds(lo, n)],
                        sem_g.at[p], add=True)

        def drain_gathers(p):
            for lo, n in SPLITS:
                for t, tab in enumerate(tables):
                    pltpu.make_async_copy(
                        tab.at[idx_v.at[p, t, pl.ds(lo, n)]],
                        acc_v.at[p, pl.ds(lo, n)],
                        sem_g.at[p]).wait()

        def fire_wb(r, p):
            pltpu.async_copy(acc_v.at[p], out_hbm.at[wid * ROWS_PW + r],
                             sem_out.at[p])

        def drain_wb(p):
            pltpu.make_async_copy(acc_v.at[p], out_hbm.at[wid * ROWS_PW],
                                  sem_out.at[p]).wait()

        fire_idx(0, 0)

        @pl.loop(0, ROWS_PW, step=2)
        def _round(rbase):
            for p in (0, 1):
                r = rbase + p
                q = p ^ 1

                @pl.when(r >= 2)
                def _():
                    drain_wb(p)

                zero_acc(p)
                wait_idx(p)
                fire_gathers(p)

                @pl.when(r >= 1)
                def _():
                    drain_gathers(q)
                    fire_wb(r - 1, q)

                fire_idx(jnp.minimum(r + 1, ROWS_PW - 1), q)

        drain_gathers(1)
        fire_wb(ROWS_PW - 1, 1)
        wait_idx(0)
        drain_wb(0)
        drain_wb(1)

    return sc(*idx_list, proj, rep)


def kernel(input_ids, token_types, n_lower, n_upper, n_alpha, n_spaces,
           n_numeric, n_special, rx_ids, ry_ids,
           we_table, we_proj_w, we_proj_b,
           t_lower, t_upper, t_alpha, t_spaces, t_numeric, t_special,
           t_types, t_rx, t_ry):
    proj = _project_table(we_table, we_proj_w, we_proj_b)
    tt_spread = jnp.tile(t_types, (128, 1))
    stacked = jnp.concatenate(
        [tt_spread.reshape(-1), t_lower.reshape(-1), t_upper.reshape(-1),
         t_alpha.reshape(-1), t_spaces.reshape(-1), t_numeric.reshape(-1),
         t_special.reshape(-1), t_rx.reshape(-1), t_ry.reshape(-1)])
    rep = _replicate(stacked, NREP)
    rep = rep.reshape(NREP, stacked.shape[0] // D, D)
    pos = jnp.arange(TOK, dtype=jnp.int32)
    tt_idx = token_types.reshape(-1).astype(jnp.int32) + 4 * (pos % 128)
    offs = [512, 1512, 2512, 3512, 4512, 5512, 6512, 7512]
    idx_list = (
        [input_ids.reshape(-1).astype(jnp.int32), tt_idx]
        + [(i.reshape(-1) + o).astype(jnp.int32) for i, o in zip(
            [n_lower, n_upper, n_alpha, n_spaces, n_numeric,
             n_special, rx_ids, ry_ids], offs)])
    out = _gather_sum(idx_list, proj, rep)
    out_t = _transpose_out(out.reshape(B, L * D))
    return jnp.transpose(out_t.reshape(L, D, B), (2, 0, 1))


# speedup vs baseline: 1.1759x; 1.1759x over previous
import functools

import jax
import jax.numpy as jnp
from jax import lax
from jax.experimental import pallas as pl
from jax.experimental.pallas import tpu as pltpu
from jax.experimental.pallas import tpu_sc as plsc

B, L = 1024, 200
TOK = B * L
D = 64
NC, NS = 2, 16
NW = NC * NS
TPW = TOK // NW
ROWS_PW = TPW // L
SPLITS = ((0, 128), (128, 72))
NREP = NW


def _project_table(we_table, we_proj_w, we_proj_b):
    V, K = we_table.shape
    Dp = we_proj_w.shape[1]
    BM = 2048

    def mm(x_ref, w_ref, b_ref, o_ref):
        o_ref[...] = (
            jnp.dot(x_ref[...], w_ref[...], preferred_element_type=jnp.float32)
            + b_ref[...]
        )

    return pl.pallas_call(
        mm,
        grid=(pl.cdiv(V, BM),),
        in_specs=[
            pl.BlockSpec((BM, K), lambda i: (i, 0)),
            pl.BlockSpec((K, Dp), lambda i: (0, 0)),
            pl.BlockSpec((1, Dp), lambda i: (0, 0)),
        ],
        out_specs=pl.BlockSpec((BM, Dp), lambda i: (i, 0)),
        out_shape=jax.ShapeDtypeStruct((V, Dp), jnp.float32),
    )(we_table, we_proj_w, we_proj_b.reshape(1, Dp))


def _replicate(flat, n):
    m = flat.shape[0]

    def cp(x_ref, o_ref):
        o_ref[...] = x_ref[...]

    return pl.pallas_call(
        cp,
        grid=(n,),
        in_specs=[pl.BlockSpec((m,), lambda i: (0,))],
        out_specs=pl.BlockSpec((m,), lambda i: (i,)),
        out_shape=jax.ShapeDtypeStruct((n * m,), jnp.float32),
    )(flat)


def _gather_sum(idx_list, proj, rep):
    mesh = plsc.VectorSubcoreMesh(core_axis_name="c", subcore_axis_name="s")

    @functools.partial(
        pl.kernel,
        out_type=jax.ShapeDtypeStruct((B, L, D), jnp.float32),
        mesh=mesh,
        scratch_types=[
            pltpu.VMEM((2, 10, L), jnp.int32),
            pltpu.VMEM((2, L, D), jnp.float32),
            pltpu.SemaphoreType.DMA((2,)),
            pltpu.SemaphoreType.DMA((2,)),
            pltpu.SemaphoreType.DMA((2,)),
        ],
        compiler_params=pltpu.CompilerParams(use_tc_tiling_on_sc=False),
    )
    def sc(i0, i1, i2, i3, i4, i5, i6, i7, i8, i9, proj_hbm, rep_hbm, out_hbm,
           idx_v, acc_v, sem_idx, sem_g, sem_out):
        wid = lax.axis_index("c") * NS + lax.axis_index("s")
        base = wid * TPW
        idx_hbms = (i0, i1, i2, i3, i4, i5, i6, i7, i8, i9)
        my_rep = rep_hbm.at[wid]
        tables = (proj_hbm,) + (my_rep,) * 9

        def fire_idx(r, p):
            off = base + r * L
            for t, ih in enumerate(idx_hbms):
                pltpu.async_copy(ih.at[pl.ds(off, L)],
                                 idx_v.at[p, t], sem_idx.at[p])

        def wait_idx(p):
            for t, ih in enumerate(idx_hbms):
                pltpu.make_async_copy(ih.at[pl.ds(base, L)],
                                      idx_v.at[p, t], sem_idx.at[p]).wait()

        def zero_acc(p):
            z = jnp.zeros((16,), jnp.float32)

            @pl.loop(0, L, unroll=8)
            def _row(i):
                for j in range(D // 16):
                    acc_v[p, i, pl.ds(j * 16, 16)] = z

        def fire_gathers(p):
            for lo, n in SPLITS:
                for t, tab in enumerate(tables):
                    pltpu.async_copy(
                        tab.at[idx_v.at[p, t, pl.ds(lo, n)]],
                        acc_v.at[p, pl.ds(lo, n)],
                        sem_g.at[p], add=True)

        def drain_gathers(p):
            for lo, n in SPLITS:
                for t, tab in enumerate(tables):
                    pltpu.make_async_copy(
                        tab.at[idx_v.at[p, t, pl.ds(lo, n)]],
                        acc_v.at[p, pl.ds(lo, n)],
                        sem_g.at[p]).wait()

        def fire_wb(r, p):
            pltpu.async_copy(acc_v.at[p], out_hbm.at[wid * ROWS_PW + r],
                             sem_out.at[p])

        def drain_wb(p):
            pltpu.make_async_copy(acc_v.at[p], out_hbm.at[wid * ROWS_PW],
                                  sem_out.at[p]).wait()

        fire_idx(0, 0)

        @pl.loop(0, ROWS_PW, step=2)
        def _round(rbase):
            for p in (0, 1):
                r = rbase + p
                q = p ^ 1

                @pl.when(r >= 2)
                def _():
                    drain_wb(p)

                zero_acc(p)
                wait_idx(p)
                fire_gathers(p)

                @pl.when(r >= 1)
                def _():
                    drain_gathers(q)
                    fire_wb(r - 1, q)

                fire_idx(jnp.minimum(r + 1, ROWS_PW - 1), q)

        drain_gathers(1)
        fire_wb(ROWS_PW - 1, 1)
        wait_idx(0)
        drain_wb(0)
        drain_wb(1)

    return sc(*idx_list, proj, rep)


def kernel(input_ids, token_types, n_lower, n_upper, n_alpha, n_spaces,
           n_numeric, n_special, rx_ids, ry_ids,
           we_table, we_proj_w, we_proj_b,
           t_lower, t_upper, t_alpha, t_spaces, t_numeric, t_special,
           t_types, t_rx, t_ry):
    proj = _project_table(we_table, we_proj_w, we_proj_b)
    tt_spread = jnp.tile(t_types, (128, 1))
    stacked = jnp.concatenate(
        [tt_spread.reshape(-1), t_lower.reshape(-1), t_upper.reshape(-1),
         t_alpha.reshape(-1), t_spaces.reshape(-1), t_numeric.reshape(-1),
         t_special.reshape(-1), t_rx.reshape(-1), t_ry.reshape(-1)])
    rep = _replicate(stacked, NREP)
    rep = rep.reshape(NREP, stacked.shape[0] // D, D)
    pos = jnp.arange(TOK, dtype=jnp.int32)
    tt_idx = token_types.reshape(-1).astype(jnp.int32) + 4 * (pos % 128)
    offs = [512, 1512, 2512, 3512, 4512, 5512, 6512, 7512]
    idx_list = (
        [input_ids.reshape(-1).astype(jnp.int32), tt_idx]
        + [(i.reshape(-1) + o).astype(jnp.int32) for i, o in zip(
            [n_lower, n_upper, n_alpha, n_spaces, n_numeric,
             n_special, rx_ids, ry_ids], offs)])
    return _gather_sum(idx_list, proj, rep)
